# Initial kernel scaffold; baseline (speedup 1.0000x reference)
import jax
import jax.numpy as jnp
from jax.experimental import pallas as pl


def kernel(logits, temperature):
    raise NotImplementedError("write your pallas kernel here")


# speedup vs baseline: 12.8757x; 12.8757x over previous
import jax
import jax.numpy as jnp
from jax.experimental import pallas as pl
from jax.experimental.pallas import tpu as pltpu

_B = 64
_V = 100000
_ROW = _B - 1
_S = 8
_L = _V // _S

_KS0 = 0
_KS1 = 42
_KS2 = _KS0 ^ _KS1 ^ 0x1BD11BDA
_ROTS = ((13, 15, 26, 6), (17, 29, 16, 24))


def _sample_kernel(temp_ref, logits_ref, out_ref):
    r = jax.lax.broadcasted_iota(jnp.uint32, (_S, _L), 0)
    c = jax.lax.broadcasted_iota(jnp.uint32, (_S, _L), 1)
    flat = r * jnp.uint32(_L) + c
    ks = (jnp.uint32(_KS0), jnp.uint32(_KS1), jnp.uint32(_KS2))
    x0 = jnp.full((_S, _L), jnp.uint32(_KS0), jnp.uint32)
    x1 = flat + jnp.uint32(_ROW * _V + _KS1)
    for i in range(5):
        for d in _ROTS[i % 2]:
            x0 = x0 + x1
            x1 = (x1 << jnp.uint32(d)) | (x1 >> jnp.uint32(32 - d))
            x1 = x0 ^ x1
        x0 = x0 + ks[(i + 1) % 3]
        x1 = x1 + ks[(i + 2) % 3] + jnp.uint32(i + 1)
    bits = x0 ^ x1
    fb = (bits >> jnp.uint32(9)) | jnp.uint32(0x3F800000)
    floats = jax.lax.bitcast_convert_type(fb, jnp.float32) - jnp.float32(1.0)
    tiny = jnp.float32(jnp.finfo(jnp.float32).tiny)
    u = jnp.maximum(tiny, floats * (jnp.float32(1.0) - tiny) + tiny)
    g = -jnp.log(-jnp.log(u))
    val = logits_ref[...] / temp_ref[0] + g
    m = jnp.max(val)
    idx = jnp.where(val == m, flat.astype(jnp.int32), jnp.int32(0x7FFFFFFF))
    out_ref[0, 0] = jnp.min(idx)


def kernel(logits, temperature):
    row = logits[_ROW].reshape(_S, _L)
    out = pl.pallas_call(
        _sample_kernel,
        out_shape=jax.ShapeDtypeStruct((1, 1), jnp.int32),
        in_specs=[
            pl.BlockSpec(memory_space=pltpu.SMEM),
            pl.BlockSpec((_S, _L), lambda: (0, 0)),
        ],
        out_specs=pl.BlockSpec(memory_space=pltpu.SMEM),
    )(temperature, row)
    return out[0, 0]


# speedup vs baseline: 23.3451x; 1.8131x over previous
import jax
import jax.numpy as jnp
from jax.experimental import pallas as pl
from jax.experimental.pallas import tpu as pltpu

_B = 64
_V = 100000
_ROW = _B - 1
_S = 8
_L = _V // _S

_KS0 = 0
_KS1 = 42
_KS2 = _KS0 ^ _KS1 ^ 0x1BD11BDA
_ROTS = ((13, 15, 26, 6), (17, 29, 16, 24))


def _sample_kernel(temp_ref, logits_hbm, out_ref, blk_vmem, sem):
    cp = pltpu.make_async_copy(
        logits_hbm.at[pl.ds(_B - _S, _S), :], blk_vmem, sem)
    cp.start()
    r = jax.lax.broadcasted_iota(jnp.uint32, (_S, _L), 0)
    c = jax.lax.broadcasted_iota(jnp.uint32, (_S, _L), 1)
    flat = r * jnp.uint32(_L) + c
    ks = (jnp.uint32(_KS0), jnp.uint32(_KS1), jnp.uint32(_KS2))
    x0 = jnp.full((_S, _L), jnp.uint32(_KS0), jnp.uint32)
    x1 = flat + jnp.uint32(_ROW * _V + _KS1)
    for i in range(5):
        for d in _ROTS[i % 2]:
            x0 = x0 + x1
            x1 = (x1 << jnp.uint32(d)) | (x1 >> jnp.uint32(32 - d))
            x1 = x0 ^ x1
        x0 = x0 + ks[(i + 1) % 3]
        x1 = x1 + ks[(i + 2) % 3] + jnp.uint32(i + 1)
    bits = x0 ^ x1
    fb = (bits >> jnp.uint32(9)) | jnp.uint32(0x3F800000)
    floats = jax.lax.bitcast_convert_type(fb, jnp.float32) - jnp.float32(1.0)
    tiny = jnp.float32(jnp.finfo(jnp.float32).tiny)
    u = jnp.maximum(tiny, floats * (jnp.float32(1.0) - tiny) + tiny)
    g = -jnp.log(-jnp.log(u))
    cp.wait()
    row = jnp.concatenate(
        [blk_vmem[_S - 1 : _S, s * _L : (s + 1) * _L] for s in range(_S)],
        axis=0)
    val = row / temp_ref[0] + g
    m = jnp.max(val)
    idx = jnp.where(val == m, flat.astype(jnp.int32), jnp.int32(0x7FFFFFFF))
    out_ref[0, 0] = jnp.min(idx)


def kernel(logits, temperature):
    out = pl.pallas_call(
        _sample_kernel,
        out_shape=jax.ShapeDtypeStruct((1, 1), jnp.int32),
        in_specs=[
            pl.BlockSpec(memory_space=pltpu.SMEM),
            pl.BlockSpec(memory_space=pl.ANY),
        ],
        out_specs=pl.BlockSpec(memory_space=pltpu.SMEM),
        scratch_shapes=[
            pltpu.VMEM((_S, _V), jnp.float32),
            pltpu.SemaphoreType.DMA,
        ],
    )(temperature, logits)
    return out[0, 0]


# speedup vs baseline: 25.1796x; 1.0786x over previous
import jax
import jax.numpy as jnp
from jax.experimental import pallas as pl
from jax.experimental.pallas import tpu as pltpu

_B = 64
_V = 100000
_ROW = _B - 1
_S = 8
_LC = 12544

_KS0 = 0
_KS1 = 42
_KS2 = _KS0 ^ _KS1 ^ 0x1BD11BDA
_ROTS = ((13, 15, 26, 6), (17, 29, 16, 24))


def _rotl(x, d):
    return (x << jnp.uint32(d)) | (x >> jnp.uint32(32 - d))


def _sample_kernel(temp_ref, logits_hbm, out_ref, blk_vmem, sem):
    cp = pltpu.make_async_copy(
        logits_hbm.at[pl.ds(_B - _S, _S), :], blk_vmem, sem)
    cp.start()
    r = jax.lax.broadcasted_iota(jnp.uint32, (_S, _LC), 0)
    c = jax.lax.broadcasted_iota(jnp.uint32, (_S, _LC), 1)
    flat = r * jnp.uint32(_LC) + c
    ks = (jnp.uint32(_KS0), jnp.uint32(_KS1), jnp.uint32(_KS2))
    x1_in = flat + jnp.uint32(_ROW * _V + _KS1)
    x0 = x1_in
    x1 = x1_in ^ _rotl(x1_in, _ROTS[0][0])
    for d in _ROTS[0][1:]:
        x0 = x0 + x1
        x1 = x0 ^ _rotl(x1, d)
    x0 = x0 + ks[1]
    x1 = x1 + ks[2] + jnp.uint32(1)
    for i in range(1, 5):
        for d in _ROTS[i % 2]:
            x0 = x0 + x1
            x1 = x0 ^ _rotl(x1, d)
        x0 = x0 + ks[(i + 1) % 3]
        x1 = x1 + ks[(i + 2) % 3] + jnp.uint32(i + 1)
    bits = x0 ^ x1
    fb = (bits >> jnp.uint32(9)) | jnp.uint32(0x3F800000)
    floats = jax.lax.bitcast_convert_type(fb, jnp.float32) - jnp.float32(1.0)
    tiny = jnp.float32(jnp.finfo(jnp.float32).tiny)
    u = jnp.maximum(tiny, floats * (jnp.float32(1.0) - tiny) + tiny)
    g = -jnp.log(-jnp.log(u))
    cp.wait()
    parts = [blk_vmem[_S - 1 : _S, s * _LC : (s + 1) * _LC]
             for s in range(_S - 1)]
    tail = blk_vmem[_S - 1 : _S, (_S - 1) * _LC : _V]
    pad = jnp.zeros((1, _S * _LC - _V), jnp.float32)
    parts.append(jnp.concatenate([tail, pad], axis=1))
    row = jnp.concatenate(parts, axis=0)
    val = jnp.where(flat < jnp.uint32(_V),
                    row / temp_ref[0] + g,
                    jnp.float32(-jnp.inf))
    m = jnp.max(val)
    idx = jnp.where(val == m, flat.astype(jnp.int32), jnp.int32(0x7FFFFFFF))
    out_ref[0, 0] = jnp.min(idx)


def kernel(logits, temperature):
    out = pl.pallas_call(
        _sample_kernel,
        out_shape=jax.ShapeDtypeStruct((1, 1), jnp.int32),
        in_specs=[
            pl.BlockSpec(memory_space=pltpu.SMEM),
            pl.BlockSpec(memory_space=pl.ANY),
        ],
        out_specs=pl.BlockSpec(memory_space=pltpu.SMEM),
        scratch_shapes=[
            pltpu.VMEM((_S, _V), jnp.float32),
            pltpu.SemaphoreType.DMA,
        ],
    )(temperature, logits)
    return out[0, 0]
